# baseline (speedup 1.0000x reference)
import jax
import jax.numpy as jnp
from jax import lax
from jax.experimental import pallas as pl
from jax.experimental.pallas import tpu as pltpu
from jax.experimental.pallas import tpu_sc as plsc

N = 10000
NP = 10240
E = 320000
D = 128
NC, NS = 2, 16
NW = NC * NS
CH = 128
EPW = 10240
NCHUNK = EPW // CH
EP = EPW * NW
BS = 1024

_SC_PARAMS = None


def _mesh():
    return plsc.VectorSubcoreMesh(
        core_axis_name="c", subcore_axis_name="s", num_cores=NC, num_subcores=NS
    )


def _sc_degree_body(col_hbm, zc_hbm, out_hbm, colv, cnt):
    c = lax.axis_index("c")
    s = lax.axis_index("s")
    wid = c * NS + s
    pltpu.sync_copy(col_hbm.at[wid], colv)
    pltpu.sync_copy(zc_hbm, cnt)
    ones = jnp.full((16,), 1.0, jnp.float32)

    def chunk(j, carry):
        idx = colv[pl.ds(j * 16, 16)]
        hi = lax.shift_right_logical(idx, 7)
        lo = lax.bitwise_and(idx, 127)
        plsc.addupdate_scatter(cnt, [hi, lo], ones)
        return carry

    lax.fori_loop(0, E // NW // 16, chunk, 0)
    pltpu.sync_copy(cnt, out_hbm.at[wid])


def _sc_degree(col2, zc):
    return pl.kernel(
        _sc_degree_body,
        out_type=jax.ShapeDtypeStruct((NW, NP // 128, 128), jnp.float32),
        mesh=_mesh(),
        scratch_types=[
            pltpu.VMEM((E // NW,), jnp.int32),
            pltpu.VMEM((NP // 128, 128), jnp.float32),
        ],
        compiler_params=pltpu.CompilerParams(needs_layout_passes=False),
    )(col2, zc)


def _sc_edge_body(y_hbm, eip_hbm, z_hbm, out_hbm,
                  idxb, rbuf0, rbuf1, agg, semg0, semg1, sems0, sems1, semi):
    c = lax.axis_index("c")
    s = lax.axis_index("s")
    wid = c * NS + s
    stripe = NP // NS

    @pl.when(c == 0)
    def _():
        pltpu.sync_copy(y_hbm.at[pl.ds(s * stripe, stripe)],
                        agg.at[pl.ds(s * stripe, stripe)])

    @pl.when(c == 1)
    def _():
        pltpu.sync_copy(z_hbm.at[pl.ds(s * stripe, stripe)],
                        agg.at[pl.ds(s * stripe, stripe)])

    pltpu.sync_copy(eip_hbm.at[wid, 0], idxb.at[0])
    plsc.subcore_barrier()
    pltpu.async_copy(y_hbm.at[idxb.at[0, 0]], rbuf0, semg0)
    pltpu.async_copy(eip_hbm.at[wid, 1], idxb.at[1], semi)

    def step(i, carry):
        j = 4 * i

        pltpu.make_async_copy(y_hbm.at[idxb.at[0, 0]], rbuf0, semg0).wait()
        pltpu.async_copy(rbuf0, agg.at[idxb.at[0, 1]], sems0, add=True)

        @pl.when(j + 2 < NCHUNK)
        def _():
            pltpu.async_copy(eip_hbm.at[wid, j + 2], idxb.at[2], semi)

        @pl.when(i > 0)
        def _():
            pltpu.make_async_copy(rbuf1, agg.at[idxb.at[3, 1]], sems1).wait()
        pltpu.make_async_copy(eip_hbm.at[wid, j + 1], idxb.at[1], semi).wait()
        pltpu.async_copy(y_hbm.at[idxb.at[1, 0]], rbuf1, semg1)

        pltpu.make_async_copy(y_hbm.at[idxb.at[1, 0]], rbuf1, semg1).wait()
        pltpu.async_copy(rbuf1, agg.at[idxb.at[1, 1]], sems1, add=True)

        @pl.when(j + 3 < NCHUNK)
        def _():
            pltpu.async_copy(eip_hbm.at[wid, j + 3], idxb.at[3], semi)
        pltpu.make_async_copy(rbuf0, agg.at[idxb.at[0, 1]], sems0).wait()

        @pl.when(j + 2 < NCHUNK)
        def _():
            pltpu.make_async_copy(eip_hbm.at[wid, j + 2], idxb.at[2], semi).wait()
            pltpu.async_copy(y_hbm.at[idxb.at[2, 0]], rbuf0, semg0)

            pltpu.make_async_copy(y_hbm.at[idxb.at[2, 0]], rbuf0, semg0).wait()
            pltpu.async_copy(rbuf0, agg.at[idxb.at[2, 1]], sems0, add=True)

            @pl.when(j + 4 < NCHUNK)
            def _():
                pltpu.async_copy(eip_hbm.at[wid, j + 4], idxb.at[0], semi)
            pltpu.make_async_copy(rbuf1, agg.at[idxb.at[1, 1]], sems1).wait()
            pltpu.make_async_copy(eip_hbm.at[wid, j + 3], idxb.at[3], semi).wait()
            pltpu.async_copy(y_hbm.at[idxb.at[3, 0]], rbuf1, semg1)

            pltpu.make_async_copy(y_hbm.at[idxb.at[3, 0]], rbuf1, semg1).wait()
            pltpu.async_copy(rbuf1, agg.at[idxb.at[3, 1]], sems1, add=True)

            @pl.when(j + 5 < NCHUNK)
            def _():
                pltpu.async_copy(eip_hbm.at[wid, j + 5], idxb.at[1], semi)
            pltpu.make_async_copy(rbuf0, agg.at[idxb.at[2, 1]], sems0).wait()

            @pl.when(j + 4 < NCHUNK)
            def _():
                pltpu.make_async_copy(eip_hbm.at[wid, j + 4], idxb.at[0], semi).wait()
                pltpu.async_copy(y_hbm.at[idxb.at[0, 0]], rbuf0, semg0)

        return carry

    lax.fori_loop(0, NCHUNK // 4, step, 0)
    pltpu.make_async_copy(rbuf1, agg.at[idxb.at[3, 1]], sems1).wait()
    plsc.subcore_barrier()
    pltpu.sync_copy(agg.at[pl.ds(s * stripe, stripe)],
                    out_hbm.at[c, pl.ds(s * stripe, stripe)])


def _sc_edge(y, eip, zbig):
    return pl.kernel(
        _sc_edge_body,
        out_type=jax.ShapeDtypeStruct((NC, NP, D), jnp.float32),
        mesh=_mesh(),
        scratch_types=[
            pltpu.VMEM((4, 2, CH), jnp.int32),
            pltpu.VMEM((CH, D), jnp.float32),
            pltpu.VMEM((CH, D), jnp.float32),
            pltpu.VMEM_SHARED((NP, D), jnp.float32),
            pltpu.SemaphoreType.DMA,
            pltpu.SemaphoreType.DMA,
            pltpu.SemaphoreType.DMA,
            pltpu.SemaphoreType.DMA,
            pltpu.SemaphoreType.DMA,
        ],
        compiler_params=pltpu.CompilerParams(needs_layout_passes=False),
    )(y, eip, zbig)


def _t0_body(cnt_ref, s_ref):
    deg = jnp.sum(cnt_ref[...], axis=0) + 1.0
    s_ref[...] = lax.rsqrt(deg)


def _t0(cnt):
    return pl.pallas_call(
        _t0_body,
        out_shape=jax.ShapeDtypeStruct((NP // 128, 128), jnp.float32),
    )(cnt)


def _t1_body(x_ref, w_ref, s_ref, y_ref):
    xw = jnp.dot(x_ref[...], w_ref[...], preferred_element_type=jnp.float32)
    y_ref[...] = xw * s_ref[...]


def _t1(x, w, s_col):
    grid = (NP // BS,)
    return pl.pallas_call(
        _t1_body,
        grid=grid,
        in_specs=[
            pl.BlockSpec((BS, D), lambda i: (i, 0)),
            pl.BlockSpec((D, D), lambda i: (0, 0)),
            pl.BlockSpec((BS, 1), lambda i: (i, 0)),
        ],
        out_specs=pl.BlockSpec((BS, D), lambda i: (i, 0)),
        out_shape=jax.ShapeDtypeStruct((NP, D), jnp.float32),
    )(x, w, s_col)


def _t2_body(a0_ref, a1_ref, s_ref, b_ref, w_ref, y_ref):
    h = (a0_ref[0] + a1_ref[0]) * s_ref[...] + b_ref[...]
    h = jnp.maximum(h, 0.0)
    hw = jnp.dot(h, w_ref[...], preferred_element_type=jnp.float32)
    y_ref[...] = hw * s_ref[...]


def _t2(agg, s_col, b, w):
    grid = (NP // BS,)
    return pl.pallas_call(
        _t2_body,
        grid=grid,
        in_specs=[
            pl.BlockSpec((1, BS, D), lambda i: (0, i, 0)),
            pl.BlockSpec((1, BS, D), lambda i: (1, i, 0)),
            pl.BlockSpec((BS, 1), lambda i: (i, 0)),
            pl.BlockSpec((1, D), lambda i: (0, 0)),
            pl.BlockSpec((D, D), lambda i: (0, 0)),
        ],
        out_specs=pl.BlockSpec((BS, D), lambda i: (i, 0)),
        out_shape=jax.ShapeDtypeStruct((NP, D), jnp.float32),
    )(agg, agg, s_col, b, w)


def _t3_body(a0_ref, a1_ref, s_ref, b_ref, z_ref):
    z_ref[...] = (a0_ref[0] + a1_ref[0]) * s_ref[...] + b_ref[...]


def _t3(agg, s_col, b):
    grid = (NP // BS,)
    return pl.pallas_call(
        _t3_body,
        grid=grid,
        in_specs=[
            pl.BlockSpec((1, BS, D), lambda i: (0, i, 0)),
            pl.BlockSpec((1, BS, D), lambda i: (1, i, 0)),
            pl.BlockSpec((BS, 1), lambda i: (i, 0)),
            pl.BlockSpec((1, D), lambda i: (0, 0)),
        ],
        out_specs=pl.BlockSpec((BS, D), lambda i: (i, 0)),
        out_shape=jax.ShapeDtypeStruct((N, D), jnp.float32),
    )(agg, agg, s_col, b)


def kernel(x, edge_index, W1, b1, W2, b2, W3, b3):
    row = edge_index[0]
    col = edge_index[1]
    ppw = (EP - E) // NW
    rpw = E // NW
    zc = jnp.zeros((NP // 128, 128), jnp.float32)
    zbig = jnp.zeros((NP, D), jnp.float32)

    cnt = _sc_degree(col.reshape(NW, rpw), zc)

    padv = jnp.broadcast_to(jnp.arange(ppw, dtype=jnp.int32)[None], (NW, ppw))
    rowp = jnp.concatenate([row.reshape(NW, rpw), padv], axis=1).reshape(
        NW, NCHUNK, CH)
    colp = jnp.concatenate([col.reshape(NW, rpw), N + padv], axis=1).reshape(
        NW, NCHUNK, CH)
    eip = jnp.stack([rowp, colp], axis=2)
    b1r = b1.reshape(1, D)
    b2r = b2.reshape(1, D)
    b3r = b3.reshape(1, D)

    s_col = _t0(cnt).reshape(NP, 1)

    y = _t1(x, W1, s_col)
    agg = _sc_edge(y, eip, zbig)
    y = _t2(agg, s_col, b1r, W2)
    agg = _sc_edge(y, eip, zbig)
    y = _t2(agg, s_col, b2r, W3)
    agg = _sc_edge(y, eip, zbig)
    return _t3(agg, s_col, b3r)


# speedup vs baseline: 1.0186x; 1.0186x over previous
import jax
import jax.numpy as jnp
from jax import lax
from jax.experimental import pallas as pl
from jax.experimental.pallas import tpu as pltpu
from jax.experimental.pallas import tpu_sc as plsc

N = 10000
NP = 10240
E = 320000
D = 128
NC, NS = 2, 16
NW = NC * NS
CH = 128
EPW = 10240
NCHUNK = EPW // CH
EP = EPW * NW
BS = 2048

_SC_PARAMS = None


def _mesh():
    return plsc.VectorSubcoreMesh(
        core_axis_name="c", subcore_axis_name="s", num_cores=NC, num_subcores=NS
    )


def _sc_degree_body(col_hbm, zc_hbm, out_hbm, colv, cnt):
    c = lax.axis_index("c")
    s = lax.axis_index("s")
    wid = c * NS + s
    pltpu.sync_copy(col_hbm.at[wid], colv)
    pltpu.sync_copy(zc_hbm, cnt)
    ones = jnp.full((16,), 1.0, jnp.float32)

    def chunk(j, carry):
        idx = colv[pl.ds(j * 16, 16)]
        hi = lax.shift_right_logical(idx, 7)
        lo = lax.bitwise_and(idx, 127)
        plsc.addupdate_scatter(cnt, [hi, lo], ones)
        return carry

    lax.fori_loop(0, E // NW // 16, chunk, 0)
    pltpu.sync_copy(cnt, out_hbm.at[wid])


def _sc_degree(col2, zc):
    return pl.kernel(
        _sc_degree_body,
        out_type=jax.ShapeDtypeStruct((NW, NP // 128, 128), jnp.float32),
        mesh=_mesh(),
        scratch_types=[
            pltpu.VMEM((E // NW,), jnp.int32),
            pltpu.VMEM((NP // 128, 128), jnp.float32),
        ],
        compiler_params=pltpu.CompilerParams(needs_layout_passes=False),
    )(col2, zc)


def _sc_edge_body(y_hbm, eip_hbm, z_hbm, out_hbm,
                  idxb, rbuf0, rbuf1, agg, semg0, semg1, sems0, sems1, semi):
    c = lax.axis_index("c")
    s = lax.axis_index("s")
    wid = c * NS + s
    stripe = NP // NS

    @pl.when(c == 0)
    def _():
        pltpu.sync_copy(y_hbm.at[pl.ds(s * stripe, stripe)],
                        agg.at[pl.ds(s * stripe, stripe)])

    @pl.when(c == 1)
    def _():
        pltpu.sync_copy(z_hbm.at[pl.ds(s * stripe, stripe)],
                        agg.at[pl.ds(s * stripe, stripe)])

    pltpu.sync_copy(eip_hbm.at[wid, 0], idxb.at[0])
    plsc.subcore_barrier()
    pltpu.async_copy(y_hbm.at[idxb.at[0, 0]], rbuf0, semg0)
    pltpu.async_copy(eip_hbm.at[wid, 1], idxb.at[1], semi)

    def step(i, carry):
        j = 4 * i

        pltpu.make_async_copy(y_hbm.at[idxb.at[0, 0]], rbuf0, semg0).wait()
        pltpu.async_copy(rbuf0, agg.at[idxb.at[0, 1]], sems0, add=True)

        @pl.when(j + 2 < NCHUNK)
        def _():
            pltpu.async_copy(eip_hbm.at[wid, j + 2], idxb.at[2], semi)

        @pl.when(i > 0)
        def _():
            pltpu.make_async_copy(rbuf1, agg.at[idxb.at[3, 1]], sems1).wait()
        pltpu.make_async_copy(eip_hbm.at[wid, j + 1], idxb.at[1], semi).wait()
        pltpu.async_copy(y_hbm.at[idxb.at[1, 0]], rbuf1, semg1)

        pltpu.make_async_copy(y_hbm.at[idxb.at[1, 0]], rbuf1, semg1).wait()
        pltpu.async_copy(rbuf1, agg.at[idxb.at[1, 1]], sems1, add=True)

        @pl.when(j + 3 < NCHUNK)
        def _():
            pltpu.async_copy(eip_hbm.at[wid, j + 3], idxb.at[3], semi)
        pltpu.make_async_copy(rbuf0, agg.at[idxb.at[0, 1]], sems0).wait()

        @pl.when(j + 2 < NCHUNK)
        def _():
            pltpu.make_async_copy(eip_hbm.at[wid, j + 2], idxb.at[2], semi).wait()
            pltpu.async_copy(y_hbm.at[idxb.at[2, 0]], rbuf0, semg0)

            pltpu.make_async_copy(y_hbm.at[idxb.at[2, 0]], rbuf0, semg0).wait()
            pltpu.async_copy(rbuf0, agg.at[idxb.at[2, 1]], sems0, add=True)

            @pl.when(j + 4 < NCHUNK)
            def _():
                pltpu.async_copy(eip_hbm.at[wid, j + 4], idxb.at[0], semi)
            pltpu.make_async_copy(rbuf1, agg.at[idxb.at[1, 1]], sems1).wait()
            pltpu.make_async_copy(eip_hbm.at[wid, j + 3], idxb.at[3], semi).wait()
            pltpu.async_copy(y_hbm.at[idxb.at[3, 0]], rbuf1, semg1)

            pltpu.make_async_copy(y_hbm.at[idxb.at[3, 0]], rbuf1, semg1).wait()
            pltpu.async_copy(rbuf1, agg.at[idxb.at[3, 1]], sems1, add=True)

            @pl.when(j + 5 < NCHUNK)
            def _():
                pltpu.async_copy(eip_hbm.at[wid, j + 5], idxb.at[1], semi)
            pltpu.make_async_copy(rbuf0, agg.at[idxb.at[2, 1]], sems0).wait()

            @pl.when(j + 4 < NCHUNK)
            def _():
                pltpu.make_async_copy(eip_hbm.at[wid, j + 4], idxb.at[0], semi).wait()
                pltpu.async_copy(y_hbm.at[idxb.at[0, 0]], rbuf0, semg0)

        return carry

    lax.fori_loop(0, NCHUNK // 4, step, 0)
    pltpu.make_async_copy(rbuf1, agg.at[idxb.at[3, 1]], sems1).wait()
    plsc.subcore_barrier()
    pltpu.sync_copy(agg.at[pl.ds(s * stripe, stripe)],
                    out_hbm.at[c, pl.ds(s * stripe, stripe)])


def _sc_edge(y, eip, zbig):
    return pl.kernel(
        _sc_edge_body,
        out_type=jax.ShapeDtypeStruct((NC, NP, D), jnp.float32),
        mesh=_mesh(),
        scratch_types=[
            pltpu.VMEM((4, 2, CH), jnp.int32),
            pltpu.VMEM((CH, D), jnp.float32),
            pltpu.VMEM((CH, D), jnp.float32),
            pltpu.VMEM_SHARED((NP, D), jnp.float32),
            pltpu.SemaphoreType.DMA,
            pltpu.SemaphoreType.DMA,
            pltpu.SemaphoreType.DMA,
            pltpu.SemaphoreType.DMA,
            pltpu.SemaphoreType.DMA,
        ],
        compiler_params=pltpu.CompilerParams(needs_layout_passes=False),
    )(y, eip, zbig)


def _t0_body(cnt_ref, s_ref):
    deg = jnp.sum(cnt_ref[...], axis=0) + 1.0
    s_ref[...] = lax.rsqrt(deg)


def _t0(cnt):
    return pl.pallas_call(
        _t0_body,
        out_shape=jax.ShapeDtypeStruct((NP // 128, 128), jnp.float32),
    )(cnt)


def _t1_body(x_ref, w_ref, s_ref, y_ref):
    xw = jnp.dot(x_ref[...], w_ref[...], preferred_element_type=jnp.float32)
    y_ref[...] = xw * s_ref[...]


def _t1(x, w, s_col):
    grid = (NP // BS,)
    return pl.pallas_call(
        _t1_body,
        grid=grid,
        in_specs=[
            pl.BlockSpec((BS, D), lambda i: (i, 0)),
            pl.BlockSpec((D, D), lambda i: (0, 0)),
            pl.BlockSpec((BS, 1), lambda i: (i, 0)),
        ],
        out_specs=pl.BlockSpec((BS, D), lambda i: (i, 0)),
        out_shape=jax.ShapeDtypeStruct((NP, D), jnp.float32),
    )(x, w, s_col)


def _t2_body(a0_ref, a1_ref, s_ref, b_ref, w_ref, y_ref):
    h = (a0_ref[0] + a1_ref[0]) * s_ref[...] + b_ref[...]
    h = jnp.maximum(h, 0.0)
    hw = jnp.dot(h, w_ref[...], preferred_element_type=jnp.float32)
    y_ref[...] = hw * s_ref[...]


def _t2(agg, s_col, b, w):
    grid = (NP // BS,)
    return pl.pallas_call(
        _t2_body,
        grid=grid,
        in_specs=[
            pl.BlockSpec((1, BS, D), lambda i: (0, i, 0)),
            pl.BlockSpec((1, BS, D), lambda i: (1, i, 0)),
            pl.BlockSpec((BS, 1), lambda i: (i, 0)),
            pl.BlockSpec((1, D), lambda i: (0, 0)),
            pl.BlockSpec((D, D), lambda i: (0, 0)),
        ],
        out_specs=pl.BlockSpec((BS, D), lambda i: (i, 0)),
        out_shape=jax.ShapeDtypeStruct((NP, D), jnp.float32),
    )(agg, agg, s_col, b, w)


def _t3_body(a0_ref, a1_ref, s_ref, b_ref, z_ref):
    z_ref[...] = (a0_ref[0] + a1_ref[0]) * s_ref[...] + b_ref[...]


def _t3(agg, s_col, b):
    grid = (NP // BS,)
    return pl.pallas_call(
        _t3_body,
        grid=grid,
        in_specs=[
            pl.BlockSpec((1, BS, D), lambda i: (0, i, 0)),
            pl.BlockSpec((1, BS, D), lambda i: (1, i, 0)),
            pl.BlockSpec((BS, 1), lambda i: (i, 0)),
            pl.BlockSpec((1, D), lambda i: (0, 0)),
        ],
        out_specs=pl.BlockSpec((BS, D), lambda i: (i, 0)),
        out_shape=jax.ShapeDtypeStruct((N, D), jnp.float32),
    )(agg, agg, s_col, b)


def kernel(x, edge_index, W1, b1, W2, b2, W3, b3):
    row = edge_index[0]
    col = edge_index[1]
    ppw = (EP - E) // NW
    rpw = E // NW
    zc = jnp.zeros((NP // 128, 128), jnp.float32)
    zbig = jnp.zeros((NP, D), jnp.float32)

    cnt = _sc_degree(col.reshape(NW, rpw), zc)

    padv = jnp.broadcast_to(jnp.arange(ppw, dtype=jnp.int32)[None], (NW, ppw))
    rowp = jnp.concatenate([row.reshape(NW, rpw), padv], axis=1).reshape(
        NW, NCHUNK, CH)
    colp = jnp.concatenate([col.reshape(NW, rpw), N + padv], axis=1).reshape(
        NW, NCHUNK, CH)
    eip = jnp.stack([rowp, colp], axis=2)
    b1r = b1.reshape(1, D)
    b2r = b2.reshape(1, D)
    b3r = b3.reshape(1, D)

    s_col = _t0(cnt).reshape(NP, 1)

    y = _t1(x, W1, s_col)
    agg = _sc_edge(y, eip, zbig)
    y = _t2(agg, s_col, b1r, W2)
    agg = _sc_edge(y, eip, zbig)
    y = _t2(agg, s_col, b2r, W3)
    agg = _sc_edge(y, eip, zbig)
    return _t3(agg, s_col, b3r)


# speedup vs baseline: 1.0311x; 1.0123x over previous
import jax
import jax.numpy as jnp
from jax import lax
from jax.experimental import pallas as pl
from jax.experimental.pallas import tpu as pltpu
from jax.experimental.pallas import tpu_sc as plsc

N = 10000
NP = 10240
E = 320000
D = 128
NC, NS = 2, 16
NW = NC * NS
CH = 128
EPW = 10240
NCHUNK = EPW // CH
EP = EPW * NW
BS = 2048

_SC_PARAMS = None


def _mesh():
    return plsc.VectorSubcoreMesh(
        core_axis_name="c", subcore_axis_name="s", num_cores=NC, num_subcores=NS
    )


def _sc_degree_body(col_hbm, zc_hbm, out_hbm, colv, cnt):
    c = lax.axis_index("c")
    s = lax.axis_index("s")
    wid = c * NS + s
    pltpu.sync_copy(col_hbm.at[wid], colv)
    pltpu.sync_copy(zc_hbm, cnt)
    ones = jnp.full((16,), 1.0, jnp.float32)

    def chunk(j, carry):
        idx = colv[pl.ds(j * 16, 16)]
        hi = lax.shift_right_logical(idx, 7)
        lo = lax.bitwise_and(idx, 127)
        plsc.addupdate_scatter(cnt, [hi, lo], ones)
        return carry

    lax.fori_loop(0, E // NW // 16, chunk, 0)
    pltpu.sync_copy(cnt, out_hbm.at[wid])


def _sc_degree(col2, zc):
    return pl.kernel(
        _sc_degree_body,
        out_type=jax.ShapeDtypeStruct((NW, NP // 128, 128), jnp.float32),
        mesh=_mesh(),
        scratch_types=[
            pltpu.VMEM((E // NW,), jnp.int32),
            pltpu.VMEM((NP // 128, 128), jnp.float32),
        ],
        compiler_params=pltpu.CompilerParams(needs_layout_passes=False),
    )(col2, zc)


def _sc_edge_body(y_hbm, eip_hbm, z_hbm, out_hbm,
                  idxb, rbuf0, rbuf1, agg, semg0, semg1, sems0, sems1, semi):
    c = lax.axis_index("c")
    s = lax.axis_index("s")
    wid = c * NS + s
    stripe = NP // NS

    pltpu.sync_copy(eip_hbm.at[wid, 0], idxb.at[0])
    pltpu.async_copy(y_hbm.at[idxb.at[0, 0]], rbuf0, semg0)
    pltpu.async_copy(eip_hbm.at[wid, 1], idxb.at[1], semi)

    @pl.when(c == 0)
    def _():
        pltpu.sync_copy(y_hbm.at[pl.ds(s * stripe, stripe)],
                        agg.at[pl.ds(s * stripe, stripe)])

    @pl.when(c == 1)
    def _():
        pltpu.sync_copy(z_hbm.at[pl.ds(s * stripe, stripe)],
                        agg.at[pl.ds(s * stripe, stripe)])

    plsc.subcore_barrier()

    def step(i, carry):
        j = 4 * i

        pltpu.make_async_copy(y_hbm.at[idxb.at[0, 0]], rbuf0, semg0).wait()
        pltpu.async_copy(rbuf0, agg.at[idxb.at[0, 1]], sems0, add=True)

        @pl.when(j + 2 < NCHUNK)
        def _():
            pltpu.async_copy(eip_hbm.at[wid, j + 2], idxb.at[2], semi)

        @pl.when(i > 0)
        def _():
            pltpu.make_async_copy(rbuf1, agg.at[idxb.at[3, 1]], sems1).wait()
        pltpu.make_async_copy(eip_hbm.at[wid, j + 1], idxb.at[1], semi).wait()
        pltpu.async_copy(y_hbm.at[idxb.at[1, 0]], rbuf1, semg1)

        pltpu.make_async_copy(y_hbm.at[idxb.at[1, 0]], rbuf1, semg1).wait()
        pltpu.async_copy(rbuf1, agg.at[idxb.at[1, 1]], sems1, add=True)

        @pl.when(j + 3 < NCHUNK)
        def _():
            pltpu.async_copy(eip_hbm.at[wid, j + 3], idxb.at[3], semi)
        pltpu.make_async_copy(rbuf0, agg.at[idxb.at[0, 1]], sems0).wait()

        @pl.when(j + 2 < NCHUNK)
        def _():
            pltpu.make_async_copy(eip_hbm.at[wid, j + 2], idxb.at[2], semi).wait()
            pltpu.async_copy(y_hbm.at[idxb.at[2, 0]], rbuf0, semg0)

            pltpu.make_async_copy(y_hbm.at[idxb.at[2, 0]], rbuf0, semg0).wait()
            pltpu.async_copy(rbuf0, agg.at[idxb.at[2, 1]], sems0, add=True)

            @pl.when(j + 4 < NCHUNK)
            def _():
                pltpu.async_copy(eip_hbm.at[wid, j + 4], idxb.at[0], semi)
            pltpu.make_async_copy(rbuf1, agg.at[idxb.at[1, 1]], sems1).wait()
            pltpu.make_async_copy(eip_hbm.at[wid, j + 3], idxb.at[3], semi).wait()
            pltpu.async_copy(y_hbm.at[idxb.at[3, 0]], rbuf1, semg1)

            pltpu.make_async_copy(y_hbm.at[idxb.at[3, 0]], rbuf1, semg1).wait()
            pltpu.async_copy(rbuf1, agg.at[idxb.at[3, 1]], sems1, add=True)

            @pl.when(j + 5 < NCHUNK)
            def _():
                pltpu.async_copy(eip_hbm.at[wid, j + 5], idxb.at[1], semi)
            pltpu.make_async_copy(rbuf0, agg.at[idxb.at[2, 1]], sems0).wait()

            @pl.when(j + 4 < NCHUNK)
            def _():
                pltpu.make_async_copy(eip_hbm.at[wid, j + 4], idxb.at[0], semi).wait()
                pltpu.async_copy(y_hbm.at[idxb.at[0, 0]], rbuf0, semg0)

        return carry

    lax.fori_loop(0, NCHUNK // 4, step, 0)
    pltpu.make_async_copy(rbuf1, agg.at[idxb.at[3, 1]], sems1).wait()
    plsc.subcore_barrier()
    pltpu.sync_copy(agg.at[pl.ds(s * stripe, stripe)],
                    out_hbm.at[c, pl.ds(s * stripe, stripe)])


def _sc_edge(y, eip, zbig):
    return pl.kernel(
        _sc_edge_body,
        out_type=jax.ShapeDtypeStruct((NC, NP, D), jnp.float32),
        mesh=_mesh(),
        scratch_types=[
            pltpu.VMEM((4, 2, CH), jnp.int32),
            pltpu.VMEM((CH, D), jnp.float32),
            pltpu.VMEM((CH, D), jnp.float32),
            pltpu.VMEM_SHARED((NP, D), jnp.float32),
            pltpu.SemaphoreType.DMA,
            pltpu.SemaphoreType.DMA,
            pltpu.SemaphoreType.DMA,
            pltpu.SemaphoreType.DMA,
            pltpu.SemaphoreType.DMA,
        ],
        compiler_params=pltpu.CompilerParams(needs_layout_passes=False),
    )(y, eip, zbig)


def _t0_body(cnt_ref, s_ref):
    deg = jnp.sum(cnt_ref[...], axis=0) + 1.0
    s_ref[...] = lax.rsqrt(deg)


def _t0(cnt):
    return pl.pallas_call(
        _t0_body,
        out_shape=jax.ShapeDtypeStruct((NP // 128, 128), jnp.float32),
    )(cnt)


def _t1_body(x_ref, w_ref, s_ref, y_ref):
    xw = jnp.dot(x_ref[...], w_ref[...], preferred_element_type=jnp.float32)
    y_ref[...] = xw * s_ref[...]


def _t1(x, w, s_col):
    grid = (NP // BS,)
    return pl.pallas_call(
        _t1_body,
        grid=grid,
        in_specs=[
            pl.BlockSpec((BS, D), lambda i: (i, 0)),
            pl.BlockSpec((D, D), lambda i: (0, 0)),
            pl.BlockSpec((BS, 1), lambda i: (i, 0)),
        ],
        out_specs=pl.BlockSpec((BS, D), lambda i: (i, 0)),
        out_shape=jax.ShapeDtypeStruct((NP, D), jnp.float32),
    )(x, w, s_col)


def _t2_body(a0_ref, a1_ref, s_ref, b_ref, w_ref, y_ref):
    h = (a0_ref[0] + a1_ref[0]) * s_ref[...] + b_ref[...]
    h = jnp.maximum(h, 0.0)
    hw = jnp.dot(h, w_ref[...], preferred_element_type=jnp.float32)
    y_ref[...] = hw * s_ref[...]


def _t2(agg, s_col, b, w):
    grid = (NP // BS,)
    return pl.pallas_call(
        _t2_body,
        grid=grid,
        in_specs=[
            pl.BlockSpec((1, BS, D), lambda i: (0, i, 0)),
            pl.BlockSpec((1, BS, D), lambda i: (1, i, 0)),
            pl.BlockSpec((BS, 1), lambda i: (i, 0)),
            pl.BlockSpec((1, D), lambda i: (0, 0)),
            pl.BlockSpec((D, D), lambda i: (0, 0)),
        ],
        out_specs=pl.BlockSpec((BS, D), lambda i: (i, 0)),
        out_shape=jax.ShapeDtypeStruct((NP, D), jnp.float32),
    )(agg, agg, s_col, b, w)


def _t3_body(a0_ref, a1_ref, s_ref, b_ref, z_ref):
    z_ref[...] = (a0_ref[0] + a1_ref[0]) * s_ref[...] + b_ref[...]


def _t3(agg, s_col, b):
    grid = (NP // BS,)
    return pl.pallas_call(
        _t3_body,
        grid=grid,
        in_specs=[
            pl.BlockSpec((1, BS, D), lambda i: (0, i, 0)),
            pl.BlockSpec((1, BS, D), lambda i: (1, i, 0)),
            pl.BlockSpec((BS, 1), lambda i: (i, 0)),
            pl.BlockSpec((1, D), lambda i: (0, 0)),
        ],
        out_specs=pl.BlockSpec((BS, D), lambda i: (i, 0)),
        out_shape=jax.ShapeDtypeStruct((N, D), jnp.float32),
    )(agg, agg, s_col, b)


def kernel(x, edge_index, W1, b1, W2, b2, W3, b3):
    row = edge_index[0]
    col = edge_index[1]
    ppw = (EP - E) // NW
    rpw = E // NW
    zc = jnp.zeros((NP // 128, 128), jnp.float32)
    zbig = jnp.zeros((NP, D), jnp.float32)

    cnt = _sc_degree(col.reshape(NW, rpw), zc)

    padv = jnp.broadcast_to(jnp.arange(ppw, dtype=jnp.int32)[None], (NW, ppw))
    rowp = jnp.concatenate([row.reshape(NW, rpw), padv], axis=1).reshape(
        NW, NCHUNK, CH)
    colp = jnp.concatenate([col.reshape(NW, rpw), N + padv], axis=1).reshape(
        NW, NCHUNK, CH)
    eip = jnp.stack([rowp, colp], axis=2)
    b1r = b1.reshape(1, D)
    b2r = b2.reshape(1, D)
    b3r = b3.reshape(1, D)

    s_col = _t0(cnt).reshape(NP, 1)

    y = _t1(x, W1, s_col)
    agg = _sc_edge(y, eip, zbig)
    y = _t2(agg, s_col, b1r, W2)
    agg = _sc_edge(y, eip, zbig)
    y = _t2(agg, s_col, b2r, W3)
    agg = _sc_edge(y, eip, zbig)
    return _t3(agg, s_col, b3r)
